# baseline (speedup 1.0000x reference)
import functools

import jax
import jax.numpy as jnp
from jax import lax
from jax.experimental import pallas as pl
from jax.experimental.pallas import tpu as pltpu
from jax.experimental.pallas import tpu_sc as plsc

N = 10000
D = 128
E = 320000
NC = 2
NS = 16
NW = NC * NS
ROWS_W = 80
EW = ROWS_W * 128
E_PAD = NW * EW
N_SH = 10240
ROWS_T = N_SH // NS
DEGW = 128

_MESH = dict(core_axis_name="c", subcore_axis_name="s")


def _deg_body(dst_hbm, ones_hbm, zeros_hbm, deg_hbm, idx_v, ones_v, deg_sh):
    c = lax.axis_index("c")
    s = lax.axis_index("s")
    w = s * NC + c
    pltpu.sync_copy(ones_hbm, ones_v)
    pltpu.sync_copy(zeros_hbm, deg_sh.at[pl.ds(s * ROWS_T, ROWS_T)])
    pltpu.sync_copy(dst_hbm.at[pl.ds(w * ROWS_W, ROWS_W)], idx_v)
    plsc.subcore_barrier()

    def body(j, carry):
        pltpu.sync_copy(ones_v, deg_sh.at[idx_v.at[j]], add=True)
        return carry

    lax.fori_loop(0, ROWS_W, body, 0)
    plsc.subcore_barrier()
    pltpu.sync_copy(deg_sh.at[pl.ds(s * ROWS_T, ROWS_T)],
                    deg_hbm.at[pl.ds((c * NS + s) * ROWS_T, ROWS_T)])


@jax.jit
def _deg(dst2d, ones8, zeros8):
    f = pl.kernel(
        _deg_body,
        out_type=jax.ShapeDtypeStruct((NC * N_SH, DEGW), jnp.float32),
        mesh=plsc.VectorSubcoreMesh(**_MESH),
        scratch_types=[
            pltpu.VMEM((ROWS_W, 128), jnp.int32),
            pltpu.VMEM((128, DEGW), jnp.float32),
            pltpu.VMEM_SHARED((N_SH, DEGW), jnp.float32),
        ],
    )
    return f(dst2d, ones8, zeros8)


AROWS = 160
BROWS = 160 - AROWS
MAXR = max(AROWS, BROWS)
TOT_ROWS = NS * (AROWS + BROWS)
SAFE_ROWS = TOT_ROWS + MAXR + 8


def _agg_body(y_hbm, sd_hbm, zeros_hbm, out_hbm,
              i0, i1, i2, i3, r0, r1, agg_sh,
              is0, is1, is2, is3, gs0, gs1):
    idxb = [i0, i1, i2, i3]
    isem = [is0, is1, is2, is3]
    rows = [r0, r1]
    gsem = [gs0, gs1]
    c = lax.axis_index("c")
    s = lax.axis_index("s")
    rows_w = jnp.where(c == 0, AROWS, BROWS)
    row0 = jnp.where(c == 0, s * AROWS, NS * AROWS + s * BROWS)

    def start_i(j, q):
        pltpu.async_copy(sd_hbm.at[row0 + j], idxb[q], isem[q])

    def wait_i(j, q):
        pltpu.make_async_copy(sd_hbm.at[row0 + j], idxb[q], isem[q]).wait()

    def start_g(j, q, b):
        for u in range(4):
            pltpu.async_copy(y_hbm.at[idxb[q].at[0].at[pl.ds(u * 32, 32)]],
                             rows[b].at[pl.ds(u * 32, 32)], gsem[b])

    def wait_g(j, q, b):
        pltpu.make_async_copy(y_hbm.at[idxb[q].at[0]], rows[b],
                              gsem[b]).wait()

    def do_s(j, q, b):
        pltpu.sync_copy(rows[b], agg_sh.at[idxb[q].at[1]], add=True)

    for k in range(ROWS_T // 128):
        pltpu.sync_copy(zeros_hbm.at[pl.ds(k * 128, 128)],
                        agg_sh.at[pl.ds(s * ROWS_T + k * 128, 128)])
    plsc.subcore_barrier()

    start_i(0, 0)
    start_i(1, 1)
    wait_i(0, 0)
    start_g(0, 0, 0)
    start_i(2, 2)
    for b4 in range(1, 4):
        j = b4
        wait_i(j, b4)
        start_g(j, b4, b4 % 2)
        wait_g(j - 1, b4 - 1, (b4 - 1) % 2)
        do_s(j - 1, b4 - 1, (b4 - 1) % 2)
        start_i(j + 2, (b4 + 2) % 4)

    def quad(qq, carry):
        for b4 in range(4):
            j = 4 * qq + b4
            wait_i(j, b4)
            start_g(j, b4, b4 % 2)
            wait_g(j - 1, (b4 - 1) % 4, (b4 - 1) % 2)
            do_s(j - 1, (b4 - 1) % 4, (b4 - 1) % 2)
            start_i(j + 2, (b4 + 2) % 4)
        return carry

    lax.fori_loop(1, rows_w // 4, quad, 0)
    jlast = jnp.maximum(rows_w - 1, 3)
    wait_g(jlast, 3, 1)
    do_s(jlast, 3, 1)
    wait_i(jlast + 1, 0)
    wait_i(jlast + 2, 1)
    plsc.subcore_barrier()
    for k in range(ROWS_T // 128):
        pltpu.sync_copy(
            agg_sh.at[pl.ds(s * ROWS_T + k * 128, 128)],
            out_hbm.at[pl.ds((c * NS + s) * ROWS_T + k * 128, 128)])


@jax.jit
def _agg(y, sd, zeros128):
    f = pl.kernel(
        _agg_body,
        out_type=jax.ShapeDtypeStruct((NC * N_SH, D), jnp.float32),
        mesh=plsc.VectorSubcoreMesh(**_MESH),
        scratch_types=[pltpu.VMEM((2, 128), jnp.int32)] * 4
        + [pltpu.VMEM((128, D), jnp.float32)] * 2 + [
            pltpu.VMEM_SHARED((N_SH, D), jnp.float32),
        ] + [pltpu.SemaphoreType.DMA] * 6,
    )
    return f(y, sd, zeros128)


_R = 1000


def _mm1_body(x_ref, w_ref, d0_ref, d1_ref, o_ref):
    dis = lax.rsqrt(d0_ref[...] + d1_ref[...] + 1.0)
    xw = lax.dot_general(x_ref[...], w_ref[...], (((1,), (1,)), ((), ())),
                         preferred_element_type=jnp.float32)
    o_ref[...] = dis * xw


@jax.jit
def _mm1(x, w, d0, d1):
    grid = (N // _R,)
    return pl.pallas_call(
        _mm1_body,
        grid=grid,
        in_specs=[
            pl.BlockSpec((_R, D), lambda i: (i, 0)),
            pl.BlockSpec((D, D), lambda i: (0, 0)),
            pl.BlockSpec((_R, 1), lambda i: (i, 0)),
            pl.BlockSpec((_R, 1), lambda i: (i, 0)),
        ],
        out_specs=pl.BlockSpec((_R, D), lambda i: (i, 0)),
        out_shape=jax.ShapeDtypeStruct((N, D), jnp.float32),
    )(x, w, d0, d1)


def _mid_body(p0_ref, p1_ref, y_ref, d0_ref, d1_ref, b_ref, a_ref, w_ref,
              o_ref):
    dis = lax.rsqrt(d0_ref[...] + d1_ref[...] + 1.0)
    h = dis * (p0_ref[...] + p1_ref[...] + y_ref[...]) + b_ref[...]
    h = jnp.where(h >= 0.0, h, a_ref[...] * h)
    o_ref[...] = dis * lax.dot_general(h, w_ref[...],
                                       (((1,), (1,)), ((), ())),
                                       preferred_element_type=jnp.float32)


@jax.jit
def _mid(p0, p1, y, d0, d1, b, a, w):
    grid = (N // _R,)
    return pl.pallas_call(
        _mid_body,
        grid=grid,
        in_specs=[
            pl.BlockSpec((_R, D), lambda i: (i, 0)),
            pl.BlockSpec((_R, D), lambda i: (i, 0)),
            pl.BlockSpec((_R, D), lambda i: (i, 0)),
            pl.BlockSpec((_R, 1), lambda i: (i, 0)),
            pl.BlockSpec((_R, 1), lambda i: (i, 0)),
            pl.BlockSpec((1, D), lambda i: (0, 0)),
            pl.BlockSpec((1, D), lambda i: (0, 0)),
            pl.BlockSpec((D, D), lambda i: (0, 0)),
        ],
        out_specs=pl.BlockSpec((_R, D), lambda i: (i, 0)),
        out_shape=jax.ShapeDtypeStruct((N, D), jnp.float32),
    )(p0, p1, y, d0, d1, b, a, w)


def _fin_body(p0_ref, p1_ref, y_ref, d0_ref, d1_ref, b_ref, a_ref, o_ref):
    dis = lax.rsqrt(d0_ref[...] + d1_ref[...] + 1.0)
    h = dis * (p0_ref[...] + p1_ref[...] + y_ref[...]) + b_ref[...]
    o_ref[...] = jnp.where(h >= 0.0, h, a_ref[...] * h)


@jax.jit
def _fin(p0, p1, y, d0, d1, b, a):
    grid = (N // _R,)
    return pl.pallas_call(
        _fin_body,
        grid=grid,
        in_specs=[
            pl.BlockSpec((_R, D), lambda i: (i, 0)),
            pl.BlockSpec((_R, D), lambda i: (i, 0)),
            pl.BlockSpec((_R, D), lambda i: (i, 0)),
            pl.BlockSpec((_R, 1), lambda i: (i, 0)),
            pl.BlockSpec((_R, 1), lambda i: (i, 0)),
            pl.BlockSpec((1, D), lambda i: (0, 0)),
            pl.BlockSpec((1, D), lambda i: (0, 0)),
        ],
        out_specs=pl.BlockSpec((_R, D), lambda i: (i, 0)),
        out_shape=jax.ShapeDtypeStruct((N, D), jnp.float32),
    )(p0, p1, y, d0, d1, b, a)


def kernel(x, edge_index, W1, b1, a1, W2, b2, a2):
    ei = edge_index.astype(jnp.int32)
    src, dst = ei[0], ei[1]
    pad = SAFE_ROWS * 128 - E
    src_p = jnp.concatenate(
        [src, jnp.zeros((pad,), jnp.int32)]).reshape(SAFE_ROWS, 128)
    padvals = N + (jnp.arange(pad, dtype=jnp.int32) % (N_SH - N))
    dst_p = jnp.concatenate([dst, padvals]).reshape(SAFE_ROWS, 128)
    sd = jnp.stack([src_p, dst_p], axis=1)

    ones8 = jnp.ones((128, DEGW), jnp.float32)
    zeros_deg = jnp.zeros((ROWS_T, DEGW), jnp.float32)
    zeros128 = zeros_deg

    degp = _deg(dst_p, ones8, zeros_deg)
    d0 = degp[0:N, 0:1]
    d1 = degp[N_SH:N_SH + N, 0:1]

    y1 = _mm1(x, W1, d0, d1)
    p = _agg(y1, sd, zeros128)
    y2 = _mid(p[0:N], p[N_SH:N_SH + N], y1, d0, d1,
              b1.reshape(1, D), a1.reshape(1, D), W2)
    q = _agg(y2, sd, zeros128)
    out = _fin(q[0:N], q[N_SH:N_SH + N], y2, d0, d1,
               b2.reshape(1, D), a2.reshape(1, D))
    return out


# speedup vs baseline: 1.9900x; 1.9900x over previous
import functools

import jax
import jax.numpy as jnp
from jax import lax
from jax.experimental import pallas as pl
from jax.experimental.pallas import tpu as pltpu
from jax.experimental.pallas import tpu_sc as plsc

N = 10000
D = 128
E = 320000
NC = 2
NS = 16
NW = NC * NS
ROWS_W = 80
EW = ROWS_W * 128
E_PAD = NW * EW
N_SH = 10240
ROWS_T = N_SH // NS
DEGW = 128

_MESH = dict(core_axis_name="c", subcore_axis_name="s")


def _deg_body(dst_hbm, ones_hbm, zeros_hbm, deg_hbm, idx_v, ones_v, deg_sh):
    c = lax.axis_index("c")
    s = lax.axis_index("s")
    w = s * NC + c
    pltpu.sync_copy(ones_hbm, ones_v)
    pltpu.sync_copy(zeros_hbm, deg_sh.at[pl.ds(s * ROWS_T, ROWS_T)])
    pltpu.sync_copy(dst_hbm.at[pl.ds(w * ROWS_W, ROWS_W)], idx_v)
    plsc.subcore_barrier()

    def body(j, carry):
        pltpu.sync_copy(ones_v, deg_sh.at[idx_v.at[j]], add=True)
        return carry

    lax.fori_loop(0, ROWS_W, body, 0)
    plsc.subcore_barrier()
    pltpu.sync_copy(deg_sh.at[pl.ds(s * ROWS_T, ROWS_T)],
                    deg_hbm.at[pl.ds((c * NS + s) * ROWS_T, ROWS_T)])


@jax.jit
def _deg(dst2d, ones8, zeros8):
    f = pl.kernel(
        _deg_body,
        out_type=jax.ShapeDtypeStruct((NC * N_SH, DEGW), jnp.float32),
        mesh=plsc.VectorSubcoreMesh(**_MESH),
        scratch_types=[
            pltpu.VMEM((ROWS_W, 128), jnp.int32),
            pltpu.VMEM((128, DEGW), jnp.float32),
            pltpu.VMEM_SHARED((N_SH, DEGW), jnp.float32),
        ],
    )
    return f(dst2d, ones8, zeros8)


AROWS = 156
BROWS = 160 - AROWS
MAXR = max(AROWS, BROWS)
TOT_ROWS = NS * (AROWS + BROWS)
SAFE_ROWS = TOT_ROWS + MAXR + 8


def _agg_body(y_hbm, sd_hbm, zeros_hbm, out_hbm,
              i0, i1, i2, i3, r0, r1, agg_sh,
              is0, is1, is2, is3, gs0, gs1):
    idxb = [i0, i1, i2, i3]
    isem = [is0, is1, is2, is3]
    rows = [r0, r1]
    gsem = [gs0, gs1]
    c = lax.axis_index("c")
    s = lax.axis_index("s")
    rows_w = jnp.where(c == 0, AROWS, BROWS)
    row0 = jnp.where(c == 0, s * AROWS, NS * AROWS + s * BROWS)

    def start_i(j, q):
        pltpu.async_copy(sd_hbm.at[row0 + j], idxb[q], isem[q])

    def wait_i(j, q):
        pltpu.make_async_copy(sd_hbm.at[row0 + j], idxb[q], isem[q]).wait()

    def start_g(j, q, b):
        for u in range(4):
            pltpu.async_copy(y_hbm.at[idxb[q].at[0].at[pl.ds(u * 32, 32)]],
                             rows[b].at[pl.ds(u * 32, 32)], gsem[b])

    def wait_g(j, q, b):
        pltpu.make_async_copy(y_hbm.at[idxb[q].at[0]], rows[b],
                              gsem[b]).wait()

    def do_s(j, q, b):
        pltpu.sync_copy(rows[b], agg_sh.at[idxb[q].at[1]], add=True)

    for k in range(ROWS_T // 128):
        pltpu.sync_copy(zeros_hbm.at[pl.ds(k * 128, 128)],
                        agg_sh.at[pl.ds(s * ROWS_T + k * 128, 128)])
    plsc.subcore_barrier()

    start_i(0, 0)
    start_i(1, 1)
    wait_i(0, 0)
    start_g(0, 0, 0)
    start_i(2, 2)
    for b4 in range(1, 4):
        j = b4
        wait_i(j, b4)
        start_g(j, b4, b4 % 2)
        wait_g(j - 1, b4 - 1, (b4 - 1) % 2)
        do_s(j - 1, b4 - 1, (b4 - 1) % 2)
        start_i(j + 2, (b4 + 2) % 4)

    def quad(qq, carry):
        for b4 in range(4):
            j = 4 * qq + b4
            wait_i(j, b4)
            start_g(j, b4, b4 % 2)
            wait_g(j - 1, (b4 - 1) % 4, (b4 - 1) % 2)
            do_s(j - 1, (b4 - 1) % 4, (b4 - 1) % 2)
            start_i(j + 2, (b4 + 2) % 4)
        return carry

    lax.fori_loop(1, rows_w // 4, quad, 0)
    jlast = jnp.maximum(rows_w - 1, 3)
    wait_g(jlast, 3, 1)
    do_s(jlast, 3, 1)
    wait_i(jlast + 1, 0)
    wait_i(jlast + 2, 1)
    plsc.subcore_barrier()
    for k in range(ROWS_T // 128):
        pltpu.sync_copy(
            agg_sh.at[pl.ds(s * ROWS_T + k * 128, 128)],
            out_hbm.at[pl.ds((c * NS + s) * ROWS_T + k * 128, 128)])


@jax.jit
def _agg(y, sd, zeros128):
    f = pl.kernel(
        _agg_body,
        out_type=jax.ShapeDtypeStruct((NC * N_SH, D), jnp.float32),
        mesh=plsc.VectorSubcoreMesh(**_MESH),
        scratch_types=[pltpu.VMEM((2, 128), jnp.int32)] * 4
        + [pltpu.VMEM((128, D), jnp.float32)] * 2 + [
            pltpu.VMEM_SHARED((N_SH, D), jnp.float32),
        ] + [pltpu.SemaphoreType.DMA] * 6,
    )
    return f(y, sd, zeros128)


_R = 1000


def _mm1_body(x_ref, w_ref, d0_ref, d1_ref, o_ref):
    dis = lax.rsqrt(d0_ref[...] + d1_ref[...] + 1.0)
    xw = lax.dot_general(x_ref[...], w_ref[...], (((1,), (1,)), ((), ())),
                         preferred_element_type=jnp.float32)
    o_ref[...] = dis * xw


@jax.jit
def _mm1(x, w, d0, d1):
    grid = (N // _R,)
    return pl.pallas_call(
        _mm1_body,
        grid=grid,
        in_specs=[
            pl.BlockSpec((_R, D), lambda i: (i, 0)),
            pl.BlockSpec((D, D), lambda i: (0, 0)),
            pl.BlockSpec((_R, 1), lambda i: (i, 0)),
            pl.BlockSpec((_R, 1), lambda i: (i, 0)),
        ],
        out_specs=pl.BlockSpec((_R, D), lambda i: (i, 0)),
        out_shape=jax.ShapeDtypeStruct((N, D), jnp.float32),
    )(x, w, d0, d1)


def _mid_body(p0_ref, p1_ref, y_ref, d0_ref, d1_ref, b_ref, a_ref, w_ref,
              o_ref):
    dis = lax.rsqrt(d0_ref[...] + d1_ref[...] + 1.0)
    h = dis * (p0_ref[...] + p1_ref[...] + y_ref[...]) + b_ref[...]
    h = jnp.where(h >= 0.0, h, a_ref[...] * h)
    o_ref[...] = dis * lax.dot_general(h, w_ref[...],
                                       (((1,), (1,)), ((), ())),
                                       preferred_element_type=jnp.float32)


@jax.jit
def _mid(p0, p1, y, d0, d1, b, a, w):
    grid = (N // _R,)
    return pl.pallas_call(
        _mid_body,
        grid=grid,
        in_specs=[
            pl.BlockSpec((_R, D), lambda i: (i, 0)),
            pl.BlockSpec((_R, D), lambda i: (i, 0)),
            pl.BlockSpec((_R, D), lambda i: (i, 0)),
            pl.BlockSpec((_R, 1), lambda i: (i, 0)),
            pl.BlockSpec((_R, 1), lambda i: (i, 0)),
            pl.BlockSpec((1, D), lambda i: (0, 0)),
            pl.BlockSpec((1, D), lambda i: (0, 0)),
            pl.BlockSpec((D, D), lambda i: (0, 0)),
        ],
        out_specs=pl.BlockSpec((_R, D), lambda i: (i, 0)),
        out_shape=jax.ShapeDtypeStruct((N, D), jnp.float32),
    )(p0, p1, y, d0, d1, b, a, w)


def _fin_body(p0_ref, p1_ref, y_ref, d0_ref, d1_ref, b_ref, a_ref, o_ref):
    dis = lax.rsqrt(d0_ref[...] + d1_ref[...] + 1.0)
    h = dis * (p0_ref[...] + p1_ref[...] + y_ref[...]) + b_ref[...]
    o_ref[...] = jnp.where(h >= 0.0, h, a_ref[...] * h)


@jax.jit
def _fin(p0, p1, y, d0, d1, b, a):
    grid = (N // _R,)
    return pl.pallas_call(
        _fin_body,
        grid=grid,
        in_specs=[
            pl.BlockSpec((_R, D), lambda i: (i, 0)),
            pl.BlockSpec((_R, D), lambda i: (i, 0)),
            pl.BlockSpec((_R, D), lambda i: (i, 0)),
            pl.BlockSpec((_R, 1), lambda i: (i, 0)),
            pl.BlockSpec((_R, 1), lambda i: (i, 0)),
            pl.BlockSpec((1, D), lambda i: (0, 0)),
            pl.BlockSpec((1, D), lambda i: (0, 0)),
        ],
        out_specs=pl.BlockSpec((_R, D), lambda i: (i, 0)),
        out_shape=jax.ShapeDtypeStruct((N, D), jnp.float32),
    )(p0, p1, y, d0, d1, b, a)


def kernel(x, edge_index, W1, b1, a1, W2, b2, a2):
    ei = edge_index.astype(jnp.int32)
    src, dst = ei[0], ei[1]
    pad = SAFE_ROWS * 128 - E
    src_p = jnp.concatenate(
        [src, jnp.zeros((pad,), jnp.int32)]).reshape(SAFE_ROWS, 128)
    padvals = N + (jnp.arange(pad, dtype=jnp.int32) % (N_SH - N))
    dst_p = jnp.concatenate([dst, padvals]).reshape(SAFE_ROWS, 128)
    sd = jnp.stack([src_p, dst_p], axis=1)

    ones8 = jnp.ones((128, DEGW), jnp.float32)
    zeros_deg = jnp.zeros((ROWS_T, DEGW), jnp.float32)
    zeros128 = zeros_deg

    degp = _deg(dst_p, ones8, zeros_deg)
    d0 = degp[0:N, 0:1]
    d1 = degp[N_SH:N_SH + N, 0:1]

    y1 = _mm1(x, W1, d0, d1)
    p = _agg(y1, sd, zeros128)
    y2 = _mid(p[0:N], p[N_SH:N_SH + N], y1, d0, d1,
              b1.reshape(1, D), a1.reshape(1, D), W2)
    q = _agg(y2, sd, zeros128)
    out = _fin(q[0:N], q[N_SH:N_SH + N], y2, d0, d1,
               b2.reshape(1, D), a2.reshape(1, D))
    return out
